# baseline (speedup 1.0000x reference)
import jax
import jax.numpy as jnp
from jax.experimental import pallas as pl
from jax.experimental.pallas import tpu as pltpu

_N_CLS = 100
_CTX_LEN = 77
_N_CTX = 4
_PREFIX = 4
_EMBED = 512
_ZEROS = _CTX_LEN - _PREFIX - _N_CTX - 1
_CB = 10


def _gather_copies(idx_ref, table_ref, prompt, gsem):
    copies = [
        pltpu.make_async_copy(
            table_ref.at[pl.ds(idx_ref[i], 1)], prompt.at[pl.ds(i, 1)], gsem
        )
        for i in range(_PREFIX)
    ]
    copies.append(
        pltpu.make_async_copy(
            table_ref.at[pl.ds(idx_ref[_CTX_LEN - 1], 1)],
            prompt.at[pl.ds(_PREFIX + 1, 1)],
            gsem,
        )
    )
    return copies


def _tc_full(table, ctx, tokens):
    def body(idx_ref, table_ref, ctx_ref, o_ref, prompt, gsem):
        i = pl.program_id(0)

        @pl.when(i == 0)
        def _start():
            for cp in _gather_copies(idx_ref, table_ref, prompt, gsem):
                cp.start()

        @pl.when(i < 2)
        def _bulk():
            dense = jnp.concatenate(
                [ctx_ref[...], jnp.zeros((_ZEROS, _EMBED), jnp.float32)],
                axis=0,
            )
            o_ref[:, pl.ds(_PREFIX, _N_CTX + _ZEROS), :] = jnp.broadcast_to(
                dense[None], (_CB, _N_CTX + _ZEROS, _EMBED)
            )

        @pl.when(i == 0)
        def _drain():
            for cp in _gather_copies(idx_ref, table_ref, prompt, gsem):
                cp.wait()

        @pl.when(i < 2)
        def _head():
            o_ref[:, pl.ds(0, _PREFIX), :] = jnp.broadcast_to(
                prompt[pl.ds(0, _PREFIX), :][None], (_CB, _PREFIX, _EMBED)
            )
            o_ref[:, pl.ds(_CTX_LEN - 1, 1), :] = jnp.broadcast_to(
                prompt[pl.ds(_PREFIX + 1, 1), :][None], (_CB, 1, _EMBED)
            )

    grid_spec = pltpu.PrefetchScalarGridSpec(
        num_scalar_prefetch=1,
        grid=(_N_CLS // _CB,),
        in_specs=[
            pl.BlockSpec(memory_space=pl.ANY),
            pl.BlockSpec((_N_CTX, _EMBED), lambda i, idx: (0, 0)),
        ],
        out_specs=pl.BlockSpec((_CB, _CTX_LEN, _EMBED), lambda i, idx: (i, 0, 0)),
        scratch_shapes=[
            pltpu.VMEM((8, _EMBED), jnp.float32),
            pltpu.SemaphoreType.DMA,
        ],
    )
    return pl.pallas_call(
        body,
        grid_spec=grid_spec,
        out_shape=jax.ShapeDtypeStruct((_N_CLS, _CTX_LEN, _EMBED), jnp.float32),
    )(tokens, table, ctx)


def kernel(token_embedding, ctx_vectors, tokenized_prompt):
    return _tc_full(token_embedding, ctx_vectors, tokenized_prompt)


# speedup vs baseline: 1.0869x; 1.0869x over previous
import jax
import jax.numpy as jnp
from jax.experimental import pallas as pl
from jax.experimental.pallas import tpu as pltpu

_N_CLS = 100
_CTX_LEN = 77
_N_CTX = 4
_PREFIX = 4
_EMBED = 512
_ZEROS = _CTX_LEN - _PREFIX - _N_CTX - 1
_CB = 50


def _gather_copies(idx_ref, table_ref, prompt, gsem):
    copies = [
        pltpu.make_async_copy(
            table_ref.at[pl.ds(idx_ref[i], 1)], prompt.at[pl.ds(i, 1)], gsem
        )
        for i in range(_PREFIX)
    ]
    copies.append(
        pltpu.make_async_copy(
            table_ref.at[pl.ds(idx_ref[_CTX_LEN - 1], 1)],
            prompt.at[pl.ds(_PREFIX + 1, 1)],
            gsem,
        )
    )
    return copies


def _tc_full(table, ctx, tokens):
    def body(idx_ref, table_ref, ctx_ref, o_ref, prompt, gsem):
        i = pl.program_id(0)

        @pl.when(i == 0)
        def _start():
            for cp in _gather_copies(idx_ref, table_ref, prompt, gsem):
                cp.start()

        @pl.when(i < 2)
        def _bulk():
            dense = jnp.concatenate(
                [ctx_ref[...], jnp.zeros((_ZEROS, _EMBED), jnp.float32)],
                axis=0,
            )
            o_ref[:, pl.ds(_PREFIX, _N_CTX + _ZEROS), :] = jnp.broadcast_to(
                dense[None], (_CB, _N_CTX + _ZEROS, _EMBED)
            )

        @pl.when(i == 0)
        def _drain():
            for cp in _gather_copies(idx_ref, table_ref, prompt, gsem):
                cp.wait()

        @pl.when(i < 2)
        def _head():
            o_ref[:, pl.ds(0, _PREFIX), :] = jnp.broadcast_to(
                prompt[pl.ds(0, _PREFIX), :][None], (_CB, _PREFIX, _EMBED)
            )
            o_ref[:, pl.ds(_CTX_LEN - 1, 1), :] = jnp.broadcast_to(
                prompt[pl.ds(_PREFIX + 1, 1), :][None], (_CB, 1, _EMBED)
            )

    grid_spec = pltpu.PrefetchScalarGridSpec(
        num_scalar_prefetch=1,
        grid=(_N_CLS // _CB,),
        in_specs=[
            pl.BlockSpec(memory_space=pl.ANY),
            pl.BlockSpec((_N_CTX, _EMBED), lambda i, idx: (0, 0)),
        ],
        out_specs=pl.BlockSpec((_CB, _CTX_LEN, _EMBED), lambda i, idx: (i, 0, 0)),
        scratch_shapes=[
            pltpu.VMEM((8, _EMBED), jnp.float32),
            pltpu.SemaphoreType.DMA,
        ],
    )
    return pl.pallas_call(
        body,
        grid_spec=grid_spec,
        out_shape=jax.ShapeDtypeStruct((_N_CLS, _CTX_LEN, _EMBED), jnp.float32),
    )(tokens, table, ctx)


def kernel(token_embedding, ctx_vectors, tokenized_prompt):
    return _tc_full(token_embedding, ctx_vectors, tokenized_prompt)


# speedup vs baseline: 1.1102x; 1.0214x over previous
import jax
import jax.numpy as jnp
from jax.experimental import pallas as pl
from jax.experimental.pallas import tpu as pltpu

_N_CLS = 100
_CTX_LEN = 77
_N_CTX = 4
_PREFIX = 4
_EMBED = 512
_ZEROS = _CTX_LEN - _PREFIX - _N_CTX - 1
_CB = 25
_SKIP = 4


def _gather_copies(idx_ref, table_ref, prompt, gsem):
    copies = [
        pltpu.make_async_copy(
            table_ref.at[pl.ds(idx_ref[i], 1)], prompt.at[pl.ds(i, 1)], gsem
        )
        for i in range(_PREFIX)
    ]
    copies.append(
        pltpu.make_async_copy(
            table_ref.at[pl.ds(idx_ref[_CTX_LEN - 1], 1)],
            prompt.at[pl.ds(_PREFIX + 1, 1)],
            gsem,
        )
    )
    return copies


def _tc_full(table, ctx, tokens):
    def body(idx_ref, table_ref, ctx_ref, o_ref, prompt, gsem):
        i = pl.program_id(0)

        @pl.when(i == 0)
        def _start():
            for cp in _gather_copies(idx_ref, table_ref, prompt, gsem):
                cp.start()

        @pl.when(i < _SKIP)
        def _bulk():
            dense = jnp.concatenate(
                [ctx_ref[...], jnp.zeros((_ZEROS, _EMBED), jnp.float32)],
                axis=0,
            )
            o_ref[:, pl.ds(_PREFIX, _N_CTX + _ZEROS), :] = jnp.broadcast_to(
                dense[None], (_CB, _N_CTX + _ZEROS, _EMBED)
            )

        @pl.when(i == 0)
        def _drain():
            for cp in _gather_copies(idx_ref, table_ref, prompt, gsem):
                cp.wait()

        @pl.when(i < _SKIP)
        def _head():
            o_ref[:, pl.ds(0, _PREFIX), :] = jnp.broadcast_to(
                prompt[pl.ds(0, _PREFIX), :][None], (_CB, _PREFIX, _EMBED)
            )
            o_ref[:, pl.ds(_CTX_LEN - 1, 1), :] = jnp.broadcast_to(
                prompt[pl.ds(_PREFIX + 1, 1), :][None], (_CB, 1, _EMBED)
            )

    grid_spec = pltpu.PrefetchScalarGridSpec(
        num_scalar_prefetch=1,
        grid=(_N_CLS // _CB,),
        in_specs=[
            pl.BlockSpec(memory_space=pl.ANY),
            pl.BlockSpec((_N_CTX, _EMBED), lambda i, idx: (0, 0)),
        ],
        out_specs=pl.BlockSpec((_CB, _CTX_LEN, _EMBED), lambda i, idx: (i, 0, 0)),
        scratch_shapes=[
            pltpu.VMEM((8, _EMBED), jnp.float32),
            pltpu.SemaphoreType.DMA,
        ],
    )
    return pl.pallas_call(
        body,
        grid_spec=grid_spec,
        out_shape=jax.ShapeDtypeStruct((_N_CLS, _CTX_LEN, _EMBED), jnp.float32),
    )(tokens, table, ctx)


def kernel(token_embedding, ctx_vectors, tokenized_prompt):
    return _tc_full(token_embedding, ctx_vectors, tokenized_prompt)
